# baseline (speedup 1.0000x reference)
import functools

import jax
import jax.numpy as jnp
from jax import lax
from jax.experimental import pallas as pl
from jax.experimental.pallas import tpu as pltpu
from jax.experimental.pallas import tpu_sc as plsc

_NUM_CORES = 2
_NUM_SUBCORES = 16
_NUM_WORKERS = _NUM_CORES * _NUM_SUBCORES

_CH = 256
_NBUF = 4
_LANES = 16


@functools.partial(jax.jit, static_argnames=("batch", "hist", "d"))
def _gather(ids_hb, table, *, batch, hist, d):
    b_cols = batch // _NUM_WORKERS
    chunks_per_h = b_cols // _CH
    n_chunks = hist * chunks_per_h
    n_rounds = n_chunks // _NBUF
    mesh = plsc.VectorSubcoreMesh(core_axis_name="c", subcore_axis_name="s")

    @functools.partial(
        pl.kernel,
        mesh=mesh,
        out_type=jax.ShapeDtypeStruct((hist, d, batch), jnp.float32),
        scratch_types=[
            pltpu.VMEM((hist * b_cols,), jnp.int32),
            [pltpu.VMEM((_CH, d), jnp.float32) for _ in range(_NBUF)],
            [pltpu.VMEM((d, _CH), jnp.float32) for _ in range(2)],
            [pltpu.SemaphoreType.DMA for _ in range(_NBUF)],
            [pltpu.SemaphoreType.DMA for _ in range(2)],
            pltpu.SemaphoreType.DMA,
        ],
        compiler_params=pltpu.CompilerParams(
            use_tc_tiling_on_sc=False, needs_layout_passes=False
        ),
    )
    def k(idx_hbm, table_hbm, out_hbm, idx_v, g, t, sem_g, sem_t, sem_i):
        wid = lax.axis_index("s") * _NUM_CORES + lax.axis_index("c")
        col0 = wid * b_cols

        for h in range(hist):
            pltpu.async_copy(
                idx_hbm.at[pl.ds(h * batch + col0, b_cols)],
                idx_v.at[pl.ds(h * b_cols, b_cols)],
                sem_i,
            )
        for h in range(hist):
            pltpu.make_async_copy(
                idx_hbm.at[pl.ds(h * batch + col0, b_cols)],
                idx_v.at[pl.ds(h * b_cols, b_cols)],
                sem_i,
            ).wait()

        def fire_gather(c, bslot):
            pltpu.async_copy(
                table_hbm.at[idx_v.at[pl.ds(c * _CH, _CH)]],
                g[bslot],
                sem_g[bslot],
            )

        def wait_gather(c, bslot):
            pltpu.make_async_copy(
                table_hbm.at[idx_v.at[pl.ds(c * _CH, _CH)]],
                g[bslot],
                sem_g[bslot],
            ).wait()

        def out_slice(h, s):
            return out_hbm.at[h, :, pl.ds(col0 + s * _CH, _CH)]

        def fire_store(h, s, tslot):
            pltpu.async_copy(t[tslot], out_slice(h, s), sem_t[tslot])

        def wait_store(h, s, tslot):
            pltpu.make_async_copy(t[tslot], out_slice(h, s), sem_t[tslot]).wait()

        def transpose(gbuf, tbuf):
            def e_body(e, carry):
                cols = jnp.full((_LANES,), e, jnp.int32)
                for s16 in range(_CH // _LANES):
                    rows = s16 * _LANES + lax.iota(jnp.int32, _LANES)
                    tbuf[e, pl.ds(s16 * _LANES, _LANES)] = plsc.load_gather(
                        gbuf, [rows, cols]
                    )
                return carry

            lax.fori_loop(0, d, e_body, 0)

        def chunk_body(j, bslot, *, first_round, refire):
            c = _NBUF * j + bslot
            h = 2 * j + (bslot // 2)
            s = bslot % 2
            tslot = bslot % 2
            wait_gather(c, bslot)
            if not (first_round and bslot < 2):
                h_prev = 2 * j + (bslot - 2) // 2
                wait_store(h_prev, s, tslot)
            transpose(g[bslot], t[tslot])
            fire_store(h, s, tslot)
            if refire:
                fire_gather(c + _NBUF, bslot)

        for bslot in range(_NBUF):
            fire_gather(bslot, bslot)
        for bslot in range(_NBUF):
            chunk_body(0, bslot, first_round=True, refire=True)

        def body(j, carry):
            for bslot in range(_NBUF):
                chunk_body(j, bslot, first_round=False, refire=True)
            return carry

        lax.fori_loop(1, n_rounds - 1, body, 0)

        for bslot in range(_NBUF):
            chunk_body(n_rounds - 1, bslot, first_round=False, refire=False)
        for bslot in range(2):
            wait_store(hist - 1, bslot, bslot)

    return k(ids_hb, table)


def kernel(token_ids, embedding_table):
    batch, hist = token_ids.shape
    vocab, d = embedding_table.shape
    ids_hb = token_ids.T.reshape(hist * batch).astype(jnp.int32)
    out_t = _gather(ids_hb, embedding_table, batch=batch, hist=hist, d=d)
    return jnp.transpose(out_t, (2, 0, 1))


# speedup vs baseline: 1.6598x; 1.6598x over previous
import functools

import jax
import jax.numpy as jnp
from jax import lax
from jax.experimental import pallas as pl
from jax.experimental.pallas import tpu as pltpu
from jax.experimental.pallas import tpu_sc as plsc

_NUM_CORES = 2
_NUM_SUBCORES = 16
_NUM_WORKERS = _NUM_CORES * _NUM_SUBCORES

_CH = 256
_NBUF = 4
_LANES = 16
_TPAD = 257


@functools.partial(jax.jit, static_argnames=("batch", "hist", "hist_pad", "d"))
def _gather(ids_flat, table, *, batch, hist, hist_pad, d):
    b_cols = batch // _NUM_WORKERS
    chunks_per_h = b_cols // _CH
    n_chunks = hist * chunks_per_h
    n_rounds = n_chunks // _NBUF
    mesh = plsc.VectorSubcoreMesh(core_axis_name="c", subcore_axis_name="s")

    @functools.partial(
        pl.kernel,
        mesh=mesh,
        out_type=jax.ShapeDtypeStruct((hist, d, batch), jnp.float32),
        scratch_types=[
            pltpu.VMEM((hist * b_cols,), jnp.int32),
            [pltpu.VMEM((_CH, d), jnp.float32) for _ in range(_NBUF)],
            [pltpu.VMEM((d, _TPAD), jnp.float32) for _ in range(2)],
            [pltpu.SemaphoreType.DMA for _ in range(_NBUF)],
            [pltpu.SemaphoreType.DMA for _ in range(2)],
            pltpu.SemaphoreType.DMA,
        ],
        compiler_params=pltpu.CompilerParams(
            use_tc_tiling_on_sc=False, needs_layout_passes=False
        ),
    )
    def k(idx_hbm, table_hbm, out_hbm, idx_v, g, t, sem_g, sem_t, sem_i):
        wid = lax.axis_index("s") * _NUM_CORES + lax.axis_index("c")
        col0 = wid * b_cols

        for h in range(hist):
            pltpu.async_copy(
                idx_hbm.at[pl.ds(h * batch + col0, b_cols)],
                idx_v.at[pl.ds(h * b_cols, b_cols)],
                sem_i,
            )
        for h in range(hist):
            pltpu.make_async_copy(
                idx_hbm.at[pl.ds(h * batch + col0, b_cols)],
                idx_v.at[pl.ds(h * b_cols, b_cols)],
                sem_i,
            ).wait()

        def fire_gather(c, bslot):
            pltpu.async_copy(
                table_hbm.at[idx_v.at[pl.ds(c * _CH, _CH)]],
                g[bslot],
                sem_g[bslot],
            )

        def wait_gather(c, bslot):
            pltpu.make_async_copy(
                table_hbm.at[idx_v.at[pl.ds(c * _CH, _CH)]],
                g[bslot],
                sem_g[bslot],
            ).wait()

        def out_slice(h, s):
            return out_hbm.at[h, :, pl.ds(col0 + s * _CH, _CH)]

        def t_src(tslot):
            return t[tslot].at[:, pl.ds(0, _CH)]

        def fire_store(h, s, tslot):
            pltpu.async_copy(t_src(tslot), out_slice(h, s), sem_t[tslot])

        def wait_store(h, s, tslot):
            pltpu.make_async_copy(
                t_src(tslot), out_slice(h, s), sem_t[tslot]
            ).wait()

        def transpose(gbuf, tbuf):
            iota = lax.iota(jnp.int32, _LANES)

            def j_body(j, carry):
                cols = jnp.full((_LANES,), j, jnp.int32)
                for e16 in range(d // _LANES):
                    plsc.store_scatter(
                        tbuf,
                        [e16 * _LANES + iota, cols],
                        gbuf[j, pl.ds(e16 * _LANES, _LANES)],
                    )
                return carry

            lax.fori_loop(0, _CH, j_body, 0)

        def chunk_body(j, bslot, *, first_round, refire):
            c = _NBUF * j + bslot
            h = 2 * j + (bslot // 2)
            s = bslot % 2
            tslot = bslot % 2
            wait_gather(c, bslot)
            if not (first_round and bslot < 2):
                h_prev = 2 * j + (bslot - 2) // 2
                wait_store(h_prev, s, tslot)
            transpose(g[bslot], t[tslot])
            fire_store(h, s, tslot)
            if refire:
                fire_gather(c + _NBUF, bslot)

        for bslot in range(_NBUF):
            fire_gather(bslot, bslot)
        for bslot in range(_NBUF):
            chunk_body(0, bslot, first_round=True, refire=True)

        def body(j, carry):
            for bslot in range(_NBUF):
                chunk_body(j, bslot, first_round=False, refire=True)
            return carry

        lax.fori_loop(1, n_rounds - 1, body, 0)

        for bslot in range(_NBUF):
            chunk_body(n_rounds - 1, bslot, first_round=False, refire=False)
        for bslot in range(2):
            wait_store(hist - 1, bslot, bslot)

    return k(ids_flat, table)


def kernel(token_ids, embedding_table):
    batch, hist = token_ids.shape
    vocab, d = embedding_table.shape
    hist_pad = (hist + 7) // 8 * 8
    ids_flat = jnp.pad(
        token_ids.T.astype(jnp.int32), ((0, hist_pad - hist), (0, 0))
    ).reshape(hist_pad * batch)
    out_t = _gather(
        ids_flat, embedding_table, batch=batch, hist=hist, hist_pad=hist_pad, d=d
    )
    return jnp.transpose(out_t, (2, 0, 1))


# speedup vs baseline: 2.1062x; 1.2690x over previous
import functools

import jax
import jax.numpy as jnp
from jax import lax
from jax.experimental import pallas as pl
from jax.experimental.pallas import tpu as pltpu
from jax.experimental.pallas import tpu_sc as plsc

_NUM_CORES = 2
_NUM_SUBCORES = 16
_NUM_WORKERS = _NUM_CORES * _NUM_SUBCORES

_CH = 256
_NBUF = 4
_LANES = 16
_TPAD = 257


@functools.partial(jax.jit, static_argnames=("batch", "hist", "hist_pad", "d"))
def _gather(ids_flat, table, *, batch, hist, hist_pad, d):
    b_cols = batch // _NUM_WORKERS
    chunks_per_h = b_cols // _CH
    n_chunks = hist * chunks_per_h
    n_rounds = n_chunks // _NBUF
    mesh = plsc.VectorSubcoreMesh(core_axis_name="c", subcore_axis_name="s")

    @functools.partial(
        pl.kernel,
        mesh=mesh,
        out_type=jax.ShapeDtypeStruct((hist, d, batch), jnp.float32),
        scratch_types=[
            pltpu.VMEM((hist * b_cols,), jnp.int32),
            [pltpu.VMEM((_CH, d), jnp.float32) for _ in range(_NBUF)],
            [pltpu.VMEM((d, _TPAD), jnp.float32) for _ in range(2)],
            [pltpu.SemaphoreType.DMA for _ in range(_NBUF)],
            [pltpu.SemaphoreType.DMA for _ in range(2)],
            pltpu.SemaphoreType.DMA,
        ],
        compiler_params=pltpu.CompilerParams(
            use_tc_tiling_on_sc=False, needs_layout_passes=False
        ),
    )
    def k(idx_hbm, table_hbm, out_hbm, idx_v, g, t, sem_g, sem_t, sem_i):
        wid = lax.axis_index("s") * _NUM_CORES + lax.axis_index("c")
        col0 = wid * b_cols

        for h in range(hist):
            pltpu.async_copy(
                idx_hbm.at[pl.ds(h * batch + col0, b_cols)],
                idx_v.at[pl.ds(h * b_cols, b_cols)],
                sem_i,
            )
        for h in range(hist):
            pltpu.make_async_copy(
                idx_hbm.at[pl.ds(h * batch + col0, b_cols)],
                idx_v.at[pl.ds(h * b_cols, b_cols)],
                sem_i,
            ).wait()

        def fire_gather(c, bslot):
            pltpu.async_copy(
                table_hbm.at[idx_v.at[pl.ds(c * _CH, _CH)]],
                g[bslot],
                sem_g[bslot],
            )

        def wait_gather(c, bslot):
            pltpu.make_async_copy(
                table_hbm.at[idx_v.at[pl.ds(c * _CH, _CH)]],
                g[bslot],
                sem_g[bslot],
            ).wait()

        def out_slice(h, s):
            return out_hbm.at[h, :, pl.ds(col0 + s * _CH, _CH)]

        def t_src(tslot):
            return t[tslot].at[:, pl.ds(0, _CH)]

        def fire_store(h, s, tslot):
            pltpu.async_copy(t_src(tslot), out_slice(h, s), sem_t[tslot])

        def wait_store(h, s, tslot):
            pltpu.make_async_copy(
                t_src(tslot), out_slice(h, s), sem_t[tslot]
            ).wait()

        def transpose(gbuf, tbuf):
            iota = lax.iota(jnp.int32, _LANES)

            @plsc.parallel_loop(0, _CH, unroll=4)
            def j_body(j):
                cols = jnp.full((_LANES,), j, jnp.int32)
                for e16 in range(d // _LANES):
                    plsc.store_scatter(
                        tbuf,
                        [e16 * _LANES + iota, cols],
                        gbuf[j, pl.ds(e16 * _LANES, _LANES)],
                    )

        def chunk_body(j, bslot, *, first_round, refire):
            c = _NBUF * j + bslot
            h = 2 * j + (bslot // 2)
            s = bslot % 2
            tslot = bslot % 2
            wait_gather(c, bslot)
            if not (first_round and bslot < 2):
                h_prev = 2 * j + (bslot - 2) // 2
                wait_store(h_prev, s, tslot)
            transpose(g[bslot], t[tslot])
            fire_store(h, s, tslot)
            if refire:
                fire_gather(c + _NBUF, bslot)

        for bslot in range(_NBUF):
            fire_gather(bslot, bslot)
        for bslot in range(_NBUF):
            chunk_body(0, bslot, first_round=True, refire=True)

        def body(j, carry):
            for bslot in range(_NBUF):
                chunk_body(j, bslot, first_round=False, refire=True)
            return carry

        lax.fori_loop(1, n_rounds - 1, body, 0)

        for bslot in range(_NBUF):
            chunk_body(n_rounds - 1, bslot, first_round=False, refire=False)
        for bslot in range(2):
            wait_store(hist - 1, bslot, bslot)

    return k(ids_flat, table)


def kernel(token_ids, embedding_table):
    batch, hist = token_ids.shape
    vocab, d = embedding_table.shape
    hist_pad = (hist + 7) // 8 * 8
    ids_flat = jnp.pad(
        token_ids.T.astype(jnp.int32), ((0, hist_pad - hist), (0, 0))
    ).reshape(hist_pad * batch)
    out_t = _gather(
        ids_flat, embedding_table, batch=batch, hist=hist, hist_pad=hist_pad, d=d
    )
    return jnp.transpose(out_t, (2, 0, 1))
